# Initial kernel scaffold; baseline (speedup 1.0000x reference)
import jax
import jax.numpy as jnp
from jax.experimental import pallas as pl


def kernel(x, xedge, W1l, b1l, W1r, W2l, b2l, W2r, Wd, bd):
    raise NotImplementedError("write your pallas kernel here")


# speedup vs baseline: 5.4974x; 5.4974x over previous
import functools

import jax
import jax.numpy as jnp
from jax import lax
from jax.experimental import pallas as pl
from jax.experimental.pallas import tpu as pltpu
from jax.experimental.pallas import tpu_sc as plsc

_N = 10000
_E = 320000
_F = 128
_NC = 2
_NT = 16
_NW = _NC * _NT
_CHUNK = 80
_STEPS = _E // (_NW * _CHUNK)
_RPT = 624
_TAIL_OFF = _RPT * _NT
_TAIL = _N - _TAIL_OFF


def _seg_body(with_deg, *refs):
    if with_deg:
        (msg, srcs, dsts, zeros128, zeros1,
         out, deg_out, srcv, dstv, rowsv, degv, acc, sem) = refs
    else:
        (msg, srcs, dsts, zeros128,
         out, srcv, dstv, rowsv, acc, sem) = refs

    c = lax.axis_index("c")
    s = lax.axis_index("s")

    row0 = pl.multiple_of(s * _RPT, 8)
    pltpu.sync_copy(zeros128.at[pl.ds(row0, _RPT)],
                    acc.at[pl.ds(row0, _RPT)])

    @pl.when(s == _NT - 1)
    def _zero_tail():
        pltpu.sync_copy(zeros128.at[pl.ds(_TAIL_OFF, _TAIL)],
                        acc.at[pl.ds(_TAIL_OFF, _TAIL)])

    if with_deg:
        pltpu.sync_copy(zeros1, degv)
    plsc.subcore_barrier()

    base0 = c * (_E // _NC) + s * (_STEPS * _CHUNK)
    if with_deg:
        ones16 = jnp.ones((16,), jnp.float32)

    def step(j, carry):
        off = pl.multiple_of(base0 + j * _CHUNK, 8)
        pltpu.sync_copy(srcs.at[pl.ds(off, _CHUNK)], srcv)
        pltpu.sync_copy(dsts.at[pl.ds(off, _CHUNK)], dstv)
        pltpu.async_copy(msg.at[srcv], rowsv, sem).wait()
        pltpu.sync_copy(rowsv, acc.at[dstv], add=True)
        if with_deg:
            for k in range(_CHUNK // 16):
                idx = dstv[pl.ds(k * 16, 16)]
                plsc.addupdate_scatter(degv, [idx], ones16)
        return carry

    lax.fori_loop(0, _STEPS, step, 0)
    plsc.subcore_barrier()

    pltpu.sync_copy(acc.at[pl.ds(row0, _RPT)],
                    out.at[c, pl.ds(row0, _RPT)])

    @pl.when(s == _NT - 1)
    def _drain_tail():
        pltpu.sync_copy(acc.at[pl.ds(_TAIL_OFF, _TAIL)],
                        out.at[c, pl.ds(_TAIL_OFF, _TAIL)])

    if with_deg:
        w = c * _NT + s
        pltpu.sync_copy(degv, deg_out.at[pl.ds(pl.multiple_of(w * _N, 8), _N)])


def _make_seg_sum(with_deg):
    mesh = plsc.VectorSubcoreMesh(core_axis_name="c", subcore_axis_name="s")
    out_type = [jax.ShapeDtypeStruct((_NC, _N, _F), jnp.float32)]
    scratch = [
        pltpu.VMEM((_CHUNK,), jnp.int32),
        pltpu.VMEM((_CHUNK,), jnp.int32),
        pltpu.VMEM((_CHUNK, _F), jnp.float32),
        pltpu.VMEM_SHARED((_N, _F), jnp.float32),
        pltpu.SemaphoreType.DMA,
    ]
    if with_deg:
        out_type.append(jax.ShapeDtypeStruct((_NW * _N,), jnp.float32))
        scratch.insert(3, pltpu.VMEM((_N,), jnp.float32))
    return pl.kernel(
        functools.partial(_seg_body, with_deg),
        out_type=out_type,
        mesh=mesh,
        scratch_types=scratch,
        compiler_params=pltpu.CompilerParams(needs_layout_passes=False),
    )


_seg_sum_deg = _make_seg_sum(True)
_seg_sum = _make_seg_sum(False)


_BLK = 2000


def _pre_body(x_ref, w_ref, b_ref, m_ref, r_ref):
    y = lax.dot_general(x_ref[...], w_ref[...], (((1,), (0,)), ((), ())),
                        preferred_element_type=jnp.float32)
    m_ref[...] = y[:, :_F]
    r_ref[...] = y[:, _F:] + b_ref[...]


def _degsum_body(deg_ref, r_ref):
    dg = jnp.sum(deg_ref[...], axis=0)
    r_ref[...] = (1.0 / jnp.maximum(dg, 1.0))[:, None]


def _mid_body(p_ref, rdeg_ref, r1_ref, w_ref, b_ref, m_ref, r_ref):
    rdeg = rdeg_ref[...]
    h1 = jnp.maximum((p_ref[0] + p_ref[1]) * rdeg + r1_ref[...], 0.0)
    y = lax.dot_general(h1, w_ref[...], (((1,), (0,)), ((), ())),
                        preferred_element_type=jnp.float32)
    m_ref[...] = y[:, :_F]
    r_ref[...] = y[:, _F:] + b_ref[...]


def _post_body(q_ref, rdeg_ref, r2_ref, wd_ref, bd_ref, h_ref, dx_ref):
    h2 = (q_ref[0] + q_ref[1]) * rdeg_ref[...] + r2_ref[...]
    h_ref[...] = h2
    dx_ref[...] = lax.dot_general(h2, wd_ref[...], (((1,), (0,)), ((), ())),
                                  preferred_element_type=jnp.float32) + bd_ref[...]


def _row_spec(width=_F):
    return pl.BlockSpec((_BLK, width), lambda i: (i, 0))


def _part_spec(width):
    return pl.BlockSpec((_NC, _BLK, width), lambda i: (0, i, 0))


def _rdeg_spec():
    return pl.BlockSpec((_BLK, 1), lambda i: (i, 0))


def _w_spec(width):
    return pl.BlockSpec((_F, width), lambda i: (0, 0))


def _b_spec():
    return pl.BlockSpec((1, _F), lambda i: (0, 0))


_GRID = _N // _BLK

_pre = pl.pallas_call(
    _pre_body,
    grid=(_GRID,),
    in_specs=[_row_spec(), _w_spec(2 * _F), _b_spec()],
    out_specs=[_row_spec(), _row_spec()],
    out_shape=[jax.ShapeDtypeStruct((_N, _F), jnp.float32)] * 2,
)

_degsum = pl.pallas_call(
    _degsum_body,
    out_shape=jax.ShapeDtypeStruct((_N, 1), jnp.float32),
)

_mid = pl.pallas_call(
    _mid_body,
    grid=(_GRID,),
    in_specs=[_part_spec(_F), _rdeg_spec(), _row_spec(),
              _w_spec(2 * _F), _b_spec()],
    out_specs=[_row_spec(), _row_spec()],
    out_shape=[jax.ShapeDtypeStruct((_N, _F), jnp.float32)] * 2,
)

_post = pl.pallas_call(
    _post_body,
    grid=(_GRID,),
    in_specs=[_part_spec(_F), _rdeg_spec(), _row_spec(),
              _w_spec(_F), _b_spec()],
    out_specs=[_row_spec(), _row_spec()],
    out_shape=[jax.ShapeDtypeStruct((_N, _F), jnp.float32)] * 2,
)


def kernel(x, xedge, W1l, b1l, W1r, W2l, b2l, W2r, Wd, bd):
    src = xedge[0]
    dst = xedge[1]
    zeros128 = jnp.zeros((_N, _F), jnp.float32)
    zeros1 = jnp.zeros((_N,), jnp.float32)

    w1 = jnp.concatenate([W1l.T, W1r.T], axis=1)
    w2 = jnp.concatenate([W2l.T, W2r.T], axis=1)

    m1, r1 = _pre(x, w1, b1l.reshape(1, _F))
    p1, deg = _seg_sum_deg(m1, src, dst, zeros128, zeros1)
    rdeg = _degsum(deg.reshape(_NW, _N))
    m2, r2 = _mid(p1, rdeg, r1, w2, b2l.reshape(1, _F))
    (p2,) = _seg_sum(m2, src, dst, zeros128)
    h2, dx = _post(p2, rdeg, r2, Wd.T, bd.reshape(1, _F))
    return (h2, dx)


# speedup vs baseline: 9.5166x; 1.7311x over previous
import functools

import jax
import jax.numpy as jnp
from jax import lax
from jax.experimental import pallas as pl
from jax.experimental.pallas import tpu as pltpu
from jax.experimental.pallas import tpu_sc as plsc

_N = 10000
_E = 320000
_F = 128
_NC = 2
_NT = 16
_NW = _NC * _NT
_CHUNK = 40
_STEPS = _E // (_NW * _CHUNK)
_K = 5
_RPT = 624
_TAIL_OFF = _RPT * _NT
_TAIL = _N - _TAIL_OFF


def _seg_body(with_deg, *refs):
    if with_deg:
        (msg, srcs, dsts, zeros128, zeros1,
         out, deg_out, degv, acc, *rest) = refs
    else:
        (msg, srcs, dsts, zeros128,
         out, acc, *rest) = refs
    srcv = rest[:_K]
    dstv = rest[_K:2 * _K]
    rows = rest[2 * _K:3 * _K]
    isem = rest[3 * _K:4 * _K]
    jsem = rest[4 * _K:5 * _K]
    gsem = rest[5 * _K:6 * _K]
    ssem = rest[6 * _K:7 * _K]

    c = lax.axis_index("c")
    s = lax.axis_index("s")

    row0 = pl.multiple_of(s * _RPT, 8)
    pltpu.sync_copy(zeros128.at[pl.ds(row0, _RPT)],
                    acc.at[pl.ds(row0, _RPT)])

    @pl.when(s == _NT - 1)
    def _zero_tail():
        pltpu.sync_copy(zeros128.at[pl.ds(_TAIL_OFF, _TAIL)],
                        acc.at[pl.ds(_TAIL_OFF, _TAIL)])

    if with_deg:
        pltpu.sync_copy(zeros1, degv)
    plsc.subcore_barrier()

    base0 = c * (_E // _NC) + s * (_STEPS * _CHUNK)
    if with_deg:
        ones16 = jnp.ones((16,), jnp.float32)
        windows = [(k * 16, None) for k in range(_CHUNK // 16)]
        if _CHUNK % 16:
            tail = _CHUNK % 16
            windows.append((_CHUNK - 16,
                            lax.iota(jnp.int32, 16) >= (16 - tail)))

    def step(g, carry):
        off = pl.multiple_of(base0 + g * (_K * _CHUNK), 8)
        idsc, jdsc = [], []
        for j in range(_K):
            o = pl.multiple_of(off + j * _CHUNK, 8)
            idsc.append(pltpu.async_copy(srcs.at[pl.ds(o, _CHUNK)],
                                         srcv[j], isem[j]))
            jdsc.append(pltpu.async_copy(dsts.at[pl.ds(o, _CHUNK)],
                                         dstv[j], jsem[j]))
        gd = []
        for j in range(_K):
            idsc[j].wait()
            gd.append(pltpu.async_copy(msg.at[srcv[j]], rows[j], gsem[j]))
        sd = []
        for j in range(_K):
            gd[j].wait()
            jdsc[j].wait()
            sd.append(pltpu.async_copy(rows[j], acc.at[dstv[j]],
                                       ssem[j], add=True))
            if with_deg:
                for o, mask in windows:
                    idx = dstv[j][pl.ds(o, 16)]
                    plsc.addupdate_scatter(degv, [idx], ones16, mask=mask)
        for j in range(_K):
            sd[j].wait()
        return carry

    lax.fori_loop(0, _STEPS // _K, step, 0)
    plsc.subcore_barrier()

    pltpu.sync_copy(acc.at[pl.ds(row0, _RPT)],
                    out.at[c, pl.ds(row0, _RPT)])

    @pl.when(s == _NT - 1)
    def _drain_tail():
        pltpu.sync_copy(acc.at[pl.ds(_TAIL_OFF, _TAIL)],
                        out.at[c, pl.ds(_TAIL_OFF, _TAIL)])

    if with_deg:
        w = c * _NT + s
        pltpu.sync_copy(degv, deg_out.at[pl.ds(pl.multiple_of(w * _N, 8), _N)])


def _make_seg_sum(with_deg):
    mesh = plsc.VectorSubcoreMesh(core_axis_name="c", subcore_axis_name="s")
    out_type = [jax.ShapeDtypeStruct((_NC, _N, _F), jnp.float32)]
    scratch = []
    if with_deg:
        out_type.append(jax.ShapeDtypeStruct((_NW * _N,), jnp.float32))
        scratch.append(pltpu.VMEM((_N,), jnp.float32))
    scratch.append(pltpu.VMEM_SHARED((_N, _F), jnp.float32))
    scratch += [pltpu.VMEM((_CHUNK,), jnp.int32) for _ in range(2 * _K)]
    scratch += [pltpu.VMEM((_CHUNK, _F), jnp.float32) for _ in range(_K)]
    scratch += [pltpu.SemaphoreType.DMA for _ in range(4 * _K)]
    return pl.kernel(
        functools.partial(_seg_body, with_deg),
        out_type=out_type,
        mesh=mesh,
        scratch_types=scratch,
        compiler_params=pltpu.CompilerParams(needs_layout_passes=False),
    )


_seg_sum_deg = _make_seg_sum(True)
_seg_sum = _make_seg_sum(False)


_BLK = 2000


def _pre_body(x_ref, w_ref, b_ref, m_ref, r_ref):
    y = lax.dot_general(x_ref[...], w_ref[...], (((1,), (0,)), ((), ())),
                        preferred_element_type=jnp.float32)
    m_ref[...] = y[:, :_F]
    r_ref[...] = y[:, _F:] + b_ref[...]


def _degsum_body(deg_ref, r_ref):
    dg = jnp.sum(deg_ref[...], axis=0)
    r_ref[...] = (1.0 / jnp.maximum(dg, 1.0))[:, None]


def _mid_body(p_ref, rdeg_ref, r1_ref, w_ref, b_ref, m_ref, r_ref):
    rdeg = rdeg_ref[...]
    h1 = jnp.maximum((p_ref[0] + p_ref[1]) * rdeg + r1_ref[...], 0.0)
    y = lax.dot_general(h1, w_ref[...], (((1,), (0,)), ((), ())),
                        preferred_element_type=jnp.float32)
    m_ref[...] = y[:, :_F]
    r_ref[...] = y[:, _F:] + b_ref[...]


def _post_body(q_ref, rdeg_ref, r2_ref, wd_ref, bd_ref, h_ref, dx_ref):
    h2 = (q_ref[0] + q_ref[1]) * rdeg_ref[...] + r2_ref[...]
    h_ref[...] = h2
    dx_ref[...] = lax.dot_general(h2, wd_ref[...], (((1,), (0,)), ((), ())),
                                  preferred_element_type=jnp.float32) + bd_ref[...]


def _row_spec(width=_F):
    return pl.BlockSpec((_BLK, width), lambda i: (i, 0))


def _part_spec(width):
    return pl.BlockSpec((_NC, _BLK, width), lambda i: (0, i, 0))


def _rdeg_spec():
    return pl.BlockSpec((_BLK, 1), lambda i: (i, 0))


def _w_spec(width):
    return pl.BlockSpec((_F, width), lambda i: (0, 0))


def _b_spec():
    return pl.BlockSpec((1, _F), lambda i: (0, 0))


_GRID = _N // _BLK

_pre = pl.pallas_call(
    _pre_body,
    grid=(_GRID,),
    in_specs=[_row_spec(), _w_spec(2 * _F), _b_spec()],
    out_specs=[_row_spec(), _row_spec()],
    out_shape=[jax.ShapeDtypeStruct((_N, _F), jnp.float32)] * 2,
)

_degsum = pl.pallas_call(
    _degsum_body,
    out_shape=jax.ShapeDtypeStruct((_N, 1), jnp.float32),
)

_mid = pl.pallas_call(
    _mid_body,
    grid=(_GRID,),
    in_specs=[_part_spec(_F), _rdeg_spec(), _row_spec(),
              _w_spec(2 * _F), _b_spec()],
    out_specs=[_row_spec(), _row_spec()],
    out_shape=[jax.ShapeDtypeStruct((_N, _F), jnp.float32)] * 2,
)

_post = pl.pallas_call(
    _post_body,
    grid=(_GRID,),
    in_specs=[_part_spec(_F), _rdeg_spec(), _row_spec(),
              _w_spec(_F), _b_spec()],
    out_specs=[_row_spec(), _row_spec()],
    out_shape=[jax.ShapeDtypeStruct((_N, _F), jnp.float32)] * 2,
)


def kernel(x, xedge, W1l, b1l, W1r, W2l, b2l, W2r, Wd, bd):
    src = xedge[0]
    dst = xedge[1]
    zeros128 = jnp.zeros((_N, _F), jnp.float32)
    zeros1 = jnp.zeros((_N,), jnp.float32)

    w1 = jnp.concatenate([W1l.T, W1r.T], axis=1)
    w2 = jnp.concatenate([W2l.T, W2r.T], axis=1)

    m1, r1 = _pre(x, w1, b1l.reshape(1, _F))
    p1, deg = _seg_sum_deg(m1, src, dst, zeros128, zeros1)
    rdeg = _degsum(deg.reshape(_NW, _N))
    m2, r2 = _mid(p1, rdeg, r1, w2, b2l.reshape(1, _F))
    (p2,) = _seg_sum(m2, src, dst, zeros128)
    h2, dx = _post(p2, rdeg, r2, Wd.T, bd.reshape(1, _F))
    return (h2, dx)


# speedup vs baseline: 10.6522x; 1.1193x over previous
import functools

import jax
import jax.numpy as jnp
from jax import lax
from jax.experimental import pallas as pl
from jax.experimental.pallas import tpu as pltpu
from jax.experimental.pallas import tpu_sc as plsc

_N = 10000
_E = 320000
_F = 128
_NC = 2
_NT = 16
_NW = _NC * _NT
_CHUNK = 40
_STEPS = _E // (_NW * _CHUNK)
_K = 5
_RPT = 624
_TAIL_OFF = _RPT * _NT
_TAIL = _N - _TAIL_OFF


def _seg_body(with_deg, *refs):
    if with_deg:
        (msg, srcs, dsts, zeros128, zeros1,
         out, deg_out, degv, acc, *rest) = refs
    else:
        (msg, srcs, dsts, zeros128,
         out, acc, *rest) = refs
    srcv = rest[:_K]
    dstv = rest[_K:2 * _K]
    rows = rest[2 * _K:3 * _K]
    isem = rest[3 * _K:4 * _K]
    jsem = rest[4 * _K:5 * _K]
    gsem = rest[5 * _K:6 * _K]
    ssem = rest[6 * _K:7 * _K]

    c = lax.axis_index("c")
    s = lax.axis_index("s")

    row0 = pl.multiple_of(s * _RPT, 8)
    pltpu.sync_copy(zeros128.at[pl.ds(row0, _RPT)],
                    acc.at[pl.ds(row0, _RPT)])

    @pl.when(s == _NT - 1)
    def _zero_tail():
        pltpu.sync_copy(zeros128.at[pl.ds(_TAIL_OFF, _TAIL)],
                        acc.at[pl.ds(_TAIL_OFF, _TAIL)])

    if with_deg:
        pltpu.sync_copy(zeros1, degv)
    plsc.subcore_barrier()

    base0 = c * (_E // _NC) + s * (_STEPS * _CHUNK)
    if with_deg:
        ones16 = jnp.ones((16,), jnp.float32)
        windows = [(k * 16, None) for k in range(_CHUNK // 16)]
        if _CHUNK % 16:
            tail = _CHUNK % 16
            windows.append((_CHUNK - 16,
                            lax.iota(jnp.int32, 16) >= (16 - tail)))

    def step(g, carry):
        off = pl.multiple_of(base0 + g * (_K * _CHUNK), 8)
        idsc, jdsc = [], []
        for j in range(_K):
            @pl.when(g > 0)
            def _drain():
                pltpu.make_async_copy(msg, rows[j], ssem[j]).wait()

            o = pl.multiple_of(off + j * _CHUNK, 8)
            idsc.append(pltpu.async_copy(srcs.at[pl.ds(o, _CHUNK)],
                                         srcv[j], isem[j]))
            jdsc.append(pltpu.async_copy(dsts.at[pl.ds(o, _CHUNK)],
                                         dstv[j], jsem[j]))
        gd = []
        for j in range(_K):
            idsc[j].wait()
            gd.append(pltpu.async_copy(msg.at[srcv[j]], rows[j], gsem[j]))
        for j in range(_K):
            gd[j].wait()
            jdsc[j].wait()
            pltpu.async_copy(rows[j], acc.at[dstv[j]], ssem[j], add=True)
            if with_deg:
                for o, mask in windows:
                    idx = dstv[j][pl.ds(o, 16)]
                    plsc.addupdate_scatter(degv, [idx], ones16, mask=mask)
        return carry

    lax.fori_loop(0, _STEPS // _K, step, 0)
    for j in range(_K):
        pltpu.make_async_copy(msg, rows[j], ssem[j]).wait()
    plsc.subcore_barrier()

    pltpu.sync_copy(acc.at[pl.ds(row0, _RPT)],
                    out.at[c, pl.ds(row0, _RPT)])

    @pl.when(s == _NT - 1)
    def _drain_tail():
        pltpu.sync_copy(acc.at[pl.ds(_TAIL_OFF, _TAIL)],
                        out.at[c, pl.ds(_TAIL_OFF, _TAIL)])

    if with_deg:
        w = c * _NT + s
        pltpu.sync_copy(degv, deg_out.at[pl.ds(pl.multiple_of(w * _N, 8), _N)])


def _make_seg_sum(with_deg):
    mesh = plsc.VectorSubcoreMesh(core_axis_name="c", subcore_axis_name="s")
    out_type = [jax.ShapeDtypeStruct((_NC, _N, _F), jnp.float32)]
    scratch = []
    if with_deg:
        out_type.append(jax.ShapeDtypeStruct((_NW * _N,), jnp.float32))
        scratch.append(pltpu.VMEM((_N,), jnp.float32))
    scratch.append(pltpu.VMEM_SHARED((_N, _F), jnp.float32))
    scratch += [pltpu.VMEM((_CHUNK,), jnp.int32) for _ in range(2 * _K)]
    scratch += [pltpu.VMEM((_CHUNK, _F), jnp.float32) for _ in range(_K)]
    scratch += [pltpu.SemaphoreType.DMA for _ in range(4 * _K)]
    return pl.kernel(
        functools.partial(_seg_body, with_deg),
        out_type=out_type,
        mesh=mesh,
        scratch_types=scratch,
        compiler_params=pltpu.CompilerParams(needs_layout_passes=False),
    )


_seg_sum_deg = _make_seg_sum(True)
_seg_sum = _make_seg_sum(False)


_BLK = 2000


def _pre_body(x_ref, w_ref, b_ref, m_ref, r_ref):
    y = lax.dot_general(x_ref[...], w_ref[...], (((1,), (0,)), ((), ())),
                        preferred_element_type=jnp.float32)
    m_ref[...] = y[:, :_F]
    r_ref[...] = y[:, _F:] + b_ref[...]


def _degsum_body(deg_ref, r_ref):
    dg = jnp.sum(deg_ref[...], axis=0)
    r_ref[...] = (1.0 / jnp.maximum(dg, 1.0))[:, None]


def _mid_body(p_ref, rdeg_ref, r1_ref, w_ref, b_ref, m_ref, r_ref):
    rdeg = rdeg_ref[...]
    h1 = jnp.maximum((p_ref[0] + p_ref[1]) * rdeg + r1_ref[...], 0.0)
    y = lax.dot_general(h1, w_ref[...], (((1,), (0,)), ((), ())),
                        preferred_element_type=jnp.float32)
    m_ref[...] = y[:, :_F]
    r_ref[...] = y[:, _F:] + b_ref[...]


def _post_body(q_ref, rdeg_ref, r2_ref, wd_ref, bd_ref, h_ref, dx_ref):
    h2 = (q_ref[0] + q_ref[1]) * rdeg_ref[...] + r2_ref[...]
    h_ref[...] = h2
    dx_ref[...] = lax.dot_general(h2, wd_ref[...], (((1,), (0,)), ((), ())),
                                  preferred_element_type=jnp.float32) + bd_ref[...]


def _row_spec(width=_F):
    return pl.BlockSpec((_BLK, width), lambda i: (i, 0))


def _part_spec(width):
    return pl.BlockSpec((_NC, _BLK, width), lambda i: (0, i, 0))


def _rdeg_spec():
    return pl.BlockSpec((_BLK, 1), lambda i: (i, 0))


def _w_spec(width):
    return pl.BlockSpec((_F, width), lambda i: (0, 0))


def _b_spec():
    return pl.BlockSpec((1, _F), lambda i: (0, 0))


_GRID = _N // _BLK

_pre = pl.pallas_call(
    _pre_body,
    grid=(_GRID,),
    in_specs=[_row_spec(), _w_spec(2 * _F), _b_spec()],
    out_specs=[_row_spec(), _row_spec()],
    out_shape=[jax.ShapeDtypeStruct((_N, _F), jnp.float32)] * 2,
)

_degsum = pl.pallas_call(
    _degsum_body,
    out_shape=jax.ShapeDtypeStruct((_N, 1), jnp.float32),
)

_mid = pl.pallas_call(
    _mid_body,
    grid=(_GRID,),
    in_specs=[_part_spec(_F), _rdeg_spec(), _row_spec(),
              _w_spec(2 * _F), _b_spec()],
    out_specs=[_row_spec(), _row_spec()],
    out_shape=[jax.ShapeDtypeStruct((_N, _F), jnp.float32)] * 2,
)

_post = pl.pallas_call(
    _post_body,
    grid=(_GRID,),
    in_specs=[_part_spec(_F), _rdeg_spec(), _row_spec(),
              _w_spec(_F), _b_spec()],
    out_specs=[_row_spec(), _row_spec()],
    out_shape=[jax.ShapeDtypeStruct((_N, _F), jnp.float32)] * 2,
)


def kernel(x, xedge, W1l, b1l, W1r, W2l, b2l, W2r, Wd, bd):
    src = xedge[0]
    dst = xedge[1]
    zeros128 = jnp.zeros((_N, _F), jnp.float32)
    zeros1 = jnp.zeros((_N,), jnp.float32)

    w1 = jnp.concatenate([W1l.T, W1r.T], axis=1)
    w2 = jnp.concatenate([W2l.T, W2r.T], axis=1)

    m1, r1 = _pre(x, w1, b1l.reshape(1, _F))
    p1, deg = _seg_sum_deg(m1, src, dst, zeros128, zeros1)
    rdeg = _degsum(deg.reshape(_NW, _N))
    m2, r2 = _mid(p1, rdeg, r1, w2, b2l.reshape(1, _F))
    (p2,) = _seg_sum(m2, src, dst, zeros128)
    h2, dx = _post(p2, rdeg, r2, Wd.T, bd.reshape(1, _F))
    return (h2, dx)
